# Initial kernel scaffold; baseline (speedup 1.0000x reference)
import jax
import jax.numpy as jnp
from jax.experimental import pallas as pl


def kernel(x_graph_1, x_graph_2, edge_index_graph_1, edge_index_graph_2, batch_graph_1, batch_graph_2, pre_W, pre_b, conv_W, conv_b, cat_W, cat_b, post_W, post_b):
    raise NotImplementedError("write your pallas kernel here")


# speedup vs baseline: 9.6859x; 9.6859x over previous
import functools

import jax
import jax.numpy as jnp
from jax import lax
from jax.experimental import pallas as pl
from jax.experimental.pallas import tpu as pltpu
from jax.experimental.pallas import tpu_sc as plsc

N = 10000
H = 128
E = 320000
NP = 10240
NC = 2
NS = 16
NW = NC * NS
EPT = E // NW
CH = 128
NFULL = EPT // CH
TAIL = EPT - NFULL * CH
RPT = NP // NS
DPT = NP // NS


def _sc_mesh():
    return plsc.VectorSubcoreMesh(core_axis_name="c", subcore_axis_name="s",
                                  num_cores=NC, num_subcores=NS)


def _deg_body(dst_hbm, out_hbm, dst_v, dst_t, ones_v, zer_v, acc):
    c = lax.axis_index("c")
    s = lax.axis_index("s")
    wid = c * NS + s
    for j in range(CH // 16):
        ones_v[pl.ds(j * 16, 16)] = jnp.full((16,), 1.0, jnp.float32)
    for j in range(DPT // 16):
        zer_v[pl.ds(j * 16, 16)] = jnp.zeros((16,), jnp.float32)
    pltpu.sync_copy(zer_v, acc.at[pl.ds(s * DPT, DPT)])
    plsc.subcore_barrier()
    e0 = wid * EPT

    def body(i, carry):
        base = e0 + i * CH
        pltpu.sync_copy(dst_hbm.at[pl.ds(base, CH)], dst_v)
        pltpu.sync_copy(ones_v, acc.at[dst_v], add=True)
        return carry

    lax.fori_loop(0, NFULL, body, 0)
    pltpu.sync_copy(dst_hbm.at[pl.ds(e0 + NFULL * CH, TAIL)], dst_t)
    pltpu.sync_copy(ones_v.at[pl.ds(0, TAIL)], acc.at[dst_t], add=True)
    plsc.subcore_barrier()
    pltpu.sync_copy(acc.at[pl.ds(s * DPT, DPT)],
                    out_hbm.at[pl.ds(c * NP + s * DPT, DPT)])


def _sc_deg(dst):
    k = pl.kernel(
        _deg_body,
        out_type=jax.ShapeDtypeStruct((2 * NP,), jnp.float32),
        mesh=_sc_mesh(),
        scratch_types=[
            pltpu.VMEM((CH,), jnp.int32),
            pltpu.VMEM((TAIL,), jnp.int32),
            pltpu.VMEM((CH,), jnp.float32),
            pltpu.VMEM((DPT,), jnp.float32),
            pltpu.VMEM_SHARED((NP,), jnp.float32),
        ],
    )
    return k(dst)


def _agg_body(hs_hbm, src_hbm, dst_hbm, zer_hbm, out_hbm,
              src_v, dst_v, src_t, dst_t, rows_v, rows_t, acc, gsem):
    c = lax.axis_index("c")
    s = lax.axis_index("s")
    wid = c * NS + s
    pltpu.sync_copy(zer_hbm, acc.at[pl.ds(s * RPT, RPT)])
    plsc.subcore_barrier()
    e0 = wid * EPT

    def body(i, carry):
        base = e0 + i * CH
        pltpu.sync_copy(src_hbm.at[pl.ds(base, CH)], src_v)
        pltpu.sync_copy(dst_hbm.at[pl.ds(base, CH)], dst_v)
        pltpu.async_copy(hs_hbm.at[src_v], rows_v, gsem).wait()
        pltpu.sync_copy(rows_v, acc.at[dst_v], add=True)
        return carry

    lax.fori_loop(0, NFULL, body, 0)
    base = e0 + NFULL * CH
    pltpu.sync_copy(src_hbm.at[pl.ds(base, TAIL)], src_t)
    pltpu.sync_copy(dst_hbm.at[pl.ds(base, TAIL)], dst_t)
    pltpu.async_copy(hs_hbm.at[src_t], rows_t, gsem).wait()
    pltpu.sync_copy(rows_t, acc.at[dst_t], add=True)
    plsc.subcore_barrier()
    pltpu.sync_copy(acc.at[pl.ds(s * RPT, RPT)],
                    out_hbm.at[pl.ds(c * NP + s * RPT, RPT)])


def _sc_agg(hs, src, dst, zer):
    k = pl.kernel(
        _agg_body,
        out_type=jax.ShapeDtypeStruct((2 * NP, H), jnp.float32),
        mesh=_sc_mesh(),
        scratch_types=[
            pltpu.VMEM((CH,), jnp.int32),
            pltpu.VMEM((CH,), jnp.int32),
            pltpu.VMEM((TAIL,), jnp.int32),
            pltpu.VMEM((TAIL,), jnp.int32),
            pltpu.VMEM((CH, H), jnp.float32),
            pltpu.VMEM((TAIL, H), jnp.float32),
            pltpu.VMEM_SHARED((NP, H), jnp.float32),
            pltpu.SemaphoreType.DMA,
        ],
    )
    return k(hs, src, dst, zer)


def _pre_body(x_ref, w_ref, b_ref, degT_ref, h_ref, inv_ref):
    y = jnp.dot(x_ref[...], w_ref[...], preferred_element_type=jnp.float32)
    y = y + b_ref[...]
    m = jnp.mean(y, axis=0, keepdims=True)
    d = y - m
    v = jnp.mean(d * d, axis=0, keepdims=True)
    h_ref[...] = jnp.maximum(d * lax.rsqrt(v + 1e-5), 0.0)
    inv_ref[...] = lax.rsqrt(degT_ref[:, 0:1] + degT_ref[:, 1:2] + 1.0)


def _pre_call(x, w, b, degT):
    return pl.pallas_call(
        _pre_body,
        out_shape=(jax.ShapeDtypeStruct((N, H), jnp.float32),
                   jax.ShapeDtypeStruct((N, 1), jnp.float32)),
    )(x, w, b, degT)


def _preagg_body(h_ref, w_ref, inv_ref, hs_ref):
    hs_ref[...] = inv_ref[...] * jnp.dot(
        h_ref[...], w_ref[...], preferred_element_type=jnp.float32)


def _preagg_call(h, w, inv):
    return pl.pallas_call(
        _preagg_body,
        out_shape=jax.ShapeDtypeStruct((N, H), jnp.float32),
    )(h, w, inv)


def _postagg_body(parts_ref, hs_ref, inv_ref, cb_ref, cw_ref, ccb_ref,
                  hold_ref, out_ref):
    ssum = parts_ref[:N, :] + parts_ref[NP:NP + N, :] + hs_ref[...]
    agg = inv_ref[...] * ssum + cb_ref[...]
    z = jnp.dot(agg, cw_ref[...], preferred_element_type=jnp.float32)
    z = z + ccb_ref[...] + hold_ref[...]
    out_ref[...] = jnp.maximum(z, 0.0)


def _postagg_call(parts, hs, inv, cb, cw, ccb, hold):
    return pl.pallas_call(
        _postagg_body,
        out_shape=jax.ShapeDtypeStruct((N, H), jnp.float32),
    )(parts, hs, inv, cb, cw, ccb, hold)


def _post_body(h_ref, w_ref, b_ref, out_ref):
    h = h_ref[...]
    for i in range(3):
        y = jnp.dot(h, w_ref[i], preferred_element_type=jnp.float32)
        y = y + b_ref[i]
        m = jnp.mean(y, axis=0, keepdims=True)
        d = y - m
        v = jnp.mean(d * d, axis=0, keepdims=True)
        h = d * lax.rsqrt(v + 1e-5)
        if i != 2:
            h = jnp.maximum(h, 0.0)
    out_ref[...] = h


def _post_call(h, w, b):
    return pl.pallas_call(
        _post_body,
        out_shape=jax.ShapeDtypeStruct((N, H), jnp.float32),
    )(h, w, b)


def kernel(x_graph_1, x_graph_2, edge_index_graph_1, edge_index_graph_2,
           batch_graph_1, batch_graph_2, pre_W, pre_b, conv_W, conv_b,
           cat_W, cat_b, post_W, post_b):
    xs = [x_graph_1, x_graph_2]
    eis = [edge_index_graph_1, edge_index_graph_2]
    zer = jnp.zeros((RPT, H), jnp.float32)

    h = [None, None]
    inv = [None, None]
    for t in range(2):
        degp = _sc_deg(eis[t][1]).reshape(2, NP)[:, :N]
        degT = degp.T
        h[t], inv[t] = _pre_call(xs[t], pre_W[-1, t],
                                 pre_b[-1, t].reshape(1, H), degT)

    L = conv_W.shape[0]
    for l in range(L):
        for t in range(2):
            hs = _preagg_call(h[t], conv_W[l, t], inv[t])
            parts = _sc_agg(hs, eis[t][0], eis[t][1], zer)
            h[t] = _postagg_call(parts, hs, inv[t],
                                 conv_b[l, t].reshape(1, H), cat_W[l, t],
                                 cat_b[l, t].reshape(1, H), h[t])

    out = [None, None]
    for t in range(2):
        out[t] = _post_call(h[t], post_W[:, t], post_b[:, t].reshape(3, 1, H))
    return jnp.stack(out)
